# Initial kernel scaffold; baseline (speedup 1.0000x reference)
import jax
import jax.numpy as jnp
from jax.experimental import pallas as pl


def kernel(inputs, table):
    raise NotImplementedError("write your pallas kernel here")


# speedup vs baseline: 3.6335x; 3.6335x over previous
import functools

import jax
import jax.numpy as jnp
from jax import lax
from jax.experimental import pallas as pl
from jax.experimental.pallas import tpu as pltpu
from jax.experimental.pallas import tpu_sc as plsc

BATCH = 4
SEQ = 8192
DIM = 1024
CHUNK = 64


def _pos_embed_kernel(table_hbm, out_hbm, buf, sem):
    info = plsc.get_sparse_core_info()
    nc, ns = info.num_cores, info.num_subcores
    nw = nc * ns
    rows_per_w = SEQ // nw
    wid = lax.axis_index("s") * nc + lax.axis_index("c")
    base = wid * rows_per_w

    def body(i, _):
        row = base + i * CHUNK
        pltpu.sync_copy(table_hbm.at[pl.ds(row, CHUNK)], buf)
        for b in range(BATCH):
            pltpu.sync_copy(buf, out_hbm.at[b, pl.ds(row, CHUNK)])
        return _

    lax.fori_loop(0, rows_per_w // CHUNK, body, 0, unroll=True)


@jax.jit
def _pos_embed(table):
    mesh = plsc.VectorSubcoreMesh(core_axis_name="c", subcore_axis_name="s")
    fn = functools.partial(
        pl.kernel,
        mesh=mesh,
        out_type=jax.ShapeDtypeStruct((BATCH, SEQ, DIM), jnp.float32),
        scratch_types=[
            pltpu.VMEM((CHUNK, DIM), jnp.float32),
            pltpu.SemaphoreType.DMA,
        ],
    )(_pos_embed_kernel)
    return fn(table)


def kernel(inputs, table):
    del inputs
    return _pos_embed(table)
